# baseline (speedup 1.0000x reference)
import functools

import jax
import jax.numpy as jnp
from jax import lax
from jax.experimental import pallas as pl
from jax.experimental.pallas import tpu as pltpu
from jax.experimental.pallas import tpu_sc as plsc

_NUM_CORES = 2
_NUM_SUBCORES = 16
_NW = _NUM_CORES * _NUM_SUBCORES
_L = 16


def _row_logic(xs):
    zero_f = jnp.zeros((_L,), jnp.float32)
    zero_i = jnp.zeros((_L,), jnp.int32)
    one_i = jnp.ones((_L,), jnp.int32)

    def get_m(a, b, c):
        p = (a > jnp.maximum(b, c)).astype(jnp.int32)
        q = (c > jnp.maximum(a, b)).astype(jnp.int32)
        return p - q

    up = xs[0:3]
    nn = xs[3:6]
    dn = xs[6:9]
    m_u = get_m(*up)
    m_n = get_m(*nn)
    m_d = get_m(*dn)
    calc = jnp.abs(m_n) * (m_u + m_d + m_n)
    s = jnp.sign(calc)
    keep_u = s == m_u
    keep_n = s == m_n
    keep_d = s == m_d
    up2 = [jnp.where(keep_u, v, zero_f) for v in up]
    nn2 = [jnp.where(keep_n, v, zero_f) for v in nn]
    dn2 = [jnp.where(keep_d, v, zero_f) for v in dn]
    idx = jnp.where(calc == 0, one_i, jnp.where(calc == 1, zero_i, 2 * one_i))

    def pick(g):
        return jnp.where(idx == 0, g[0], jnp.where(idx == 1, g[1], g[2]))

    val_u = pick(up2)
    val_n = pick(nn2)
    val_d = pick(dn2)
    w_u = (val_u >= val_n) & (val_u >= val_d)
    w_n = jnp.logical_not(w_u) & (val_n >= val_d)
    return [jnp.where(w_u, up2[j], jnp.where(w_n, nn2[j], dn2[j]))
            for j in range(3)]


def _make_sc_kernel(n_rows):
    rows_per_w = n_rows // _NW
    chunks = rows_per_w // _L
    mesh = plsc.VectorSubcoreMesh(
        core_axis_name="c", subcore_axis_name="s", num_cores=_NUM_CORES
    )

    half = rows_per_w // 2
    hchunks = half // _L

    @functools.partial(
        pl.kernel,
        out_type=jax.ShapeDtypeStruct((3, n_rows), jnp.float32),
        mesh=mesh,
        scratch_types=[
            pltpu.VMEM((9, half), jnp.float32),
            pltpu.VMEM((9, half), jnp.float32),
            pltpu.VMEM((3, half), jnp.float32),
            pltpu.VMEM((3, half), jnp.float32),
            pltpu.SemaphoreType.DMA,
            pltpu.SemaphoreType.DMA,
            pltpu.SemaphoreType.DMA,
            pltpu.SemaphoreType.DMA,
        ],
        compiler_params=pltpu.CompilerParams(
            needs_layout_passes=False,
        ),
    )
    def sc_kernel(x_hbm, out_hbm, x_v0, x_v1, out_v0, out_v1,
                  ld0, ld1, st0, st1):
        wid = lax.axis_index("s") * _NUM_CORES + lax.axis_index("c")
        base = wid * rows_per_w

        cp0 = pltpu.make_async_copy(
            x_hbm.at[:, pl.ds(base, half)], x_v0, ld0)
        cp1 = pltpu.make_async_copy(
            x_hbm.at[:, pl.ds(base + half, half)], x_v1, ld1)
        cp0.start()
        cp1.start()

        def compute(x_v, out_v):
            @plsc.parallel_loop(0, hchunks, 1, unroll=8)
            def body(i):
                sl = pl.ds(i * _L, _L)
                xs = [x_v[c, sl] for c in range(9)]
                outs = _row_logic(xs)
                for j in range(3):
                    out_v[j, sl] = outs[j]

        cp0.wait()
        compute(x_v0, out_v0)
        w0 = pltpu.make_async_copy(
            out_v0, out_hbm.at[:, pl.ds(base, half)], st0)
        w0.start()
        cp1.wait()
        compute(x_v1, out_v1)
        w1 = pltpu.make_async_copy(
            out_v1, out_hbm.at[:, pl.ds(base + half, half)], st1)
        w1.start()
        w0.wait()
        w1.wait()

    return sc_kernel


def kernel(inputs):
    n_rows, n_feat = inputs.shape
    assert n_feat == 9 and n_rows % (_NW * _L) == 0
    out_t = _make_sc_kernel(n_rows)(inputs.T)
    return out_t.T


# speedup vs baseline: 1.1475x; 1.1475x over previous
import functools

import jax
import jax.numpy as jnp
from jax import lax
from jax.experimental import pallas as pl
from jax.experimental.pallas import tpu as pltpu
from jax.experimental.pallas import tpu_sc as plsc

_NUM_CORES = 2
_NUM_SUBCORES = 16
_NW = _NUM_CORES * _NUM_SUBCORES
_L = 16


def _row_logic(xs):
    zero_f = jnp.zeros((_L,), jnp.float32)
    zero_i = jnp.zeros((_L,), jnp.int32)
    one_i = jnp.ones((_L,), jnp.int32)

    def get_m(a, b, c):
        p = (a > jnp.maximum(b, c)).astype(jnp.int32)
        q = (c > jnp.maximum(a, b)).astype(jnp.int32)
        return p - q

    up = xs[0:3]
    nn = xs[3:6]
    dn = xs[6:9]
    m_u = get_m(*up)
    m_n = get_m(*nn)
    m_d = get_m(*dn)
    calc = jnp.abs(m_n) * (m_u + m_d + m_n)
    s = jnp.sign(calc)
    keep_u = s == m_u
    keep_n = s == m_n
    keep_d = s == m_d
    up2 = [jnp.where(keep_u, v, zero_f) for v in up]
    nn2 = [jnp.where(keep_n, v, zero_f) for v in nn]
    dn2 = [jnp.where(keep_d, v, zero_f) for v in dn]
    idx = jnp.where(calc == 0, one_i, jnp.where(calc == 1, zero_i, 2 * one_i))

    def pick(g):
        return jnp.where(idx == 0, g[0], jnp.where(idx == 1, g[1], g[2]))

    val_u = pick(up2)
    val_n = pick(nn2)
    val_d = pick(dn2)
    w_u = (val_u >= val_n) & (val_u >= val_d)
    w_n = jnp.logical_not(w_u) & (val_n >= val_d)
    return [jnp.where(w_u, up2[j], jnp.where(w_n, nn2[j], dn2[j]))
            for j in range(3)]


def _make_sc_kernel(n_rows):
    rows_per_w = n_rows // _NW
    chunks = rows_per_w // _L
    mesh = plsc.VectorSubcoreMesh(
        core_axis_name="c", subcore_axis_name="s", num_cores=_NUM_CORES
    )

    @functools.partial(
        pl.kernel,
        out_type=jax.ShapeDtypeStruct((3, n_rows), jnp.float32),
        mesh=mesh,
        scratch_types=[
            pltpu.VMEM((9, rows_per_w), jnp.float32),
            pltpu.VMEM((3, rows_per_w), jnp.float32),
        ],
        compiler_params=pltpu.CompilerParams(
            needs_layout_passes=False,
        ),
    )
    def sc_kernel(x_hbm, out_hbm, x_v, out_v):
        wid = lax.axis_index("s") * _NUM_CORES + lax.axis_index("c")
        base = wid * rows_per_w
        pltpu.sync_copy(x_hbm.at[:, pl.ds(base, rows_per_w)], x_v)

        @plsc.parallel_loop(0, chunks, 1, unroll=2)
        def body(i):
            sl = pl.ds(i * _L, _L)
            xs = [x_v[c, sl] for c in range(9)]
            outs = _row_logic(xs)
            for j in range(3):
                out_v[j, sl] = outs[j]

        pltpu.sync_copy(out_v, out_hbm.at[:, pl.ds(base, rows_per_w)])

    return sc_kernel


def kernel(inputs):
    n_rows, n_feat = inputs.shape
    assert n_feat == 9 and n_rows % (_NW * _L) == 0
    out_t = _make_sc_kernel(n_rows)(inputs.T)
    return out_t.T
